# baseline (speedup 1.0000x reference)
import jax
import jax.numpy as jnp
from jax import lax
from jax.experimental import pallas as pl
from jax.experimental.pallas import tpu as pltpu
from jax.experimental.pallas import tpu_sc as plsc

N = 10000
E = 320000
D = 128

NC = 2
NS = 16
NW = NC * NS
CHUNK = 80
NCHUNK = 128
EPT = NCHUNK * CHUNK
BLK = 8
NBLK = NCHUNK // BLK
NPAD = 10240
RPT = NPAD // NS

_mesh = plsc.VectorSubcoreMesh(core_axis_name="c", subcore_axis_name="s",
                               num_cores=NC, num_subcores=NS)


def _zero_buf(buf, nrows, ncols):
    zeros16 = jnp.zeros((16,), jnp.float32)

    def body(i, _):
        for j in range(ncols // 16):
            buf[i, pl.ds(j * 16, 16)] = zeros16
        return 0

    lax.fori_loop(0, nrows, body, 0)


def _deg_body(dst_hbm, deg_hbm, deg_sp, dst_v, ones_v):
    c = lax.axis_index("c")
    s = lax.axis_index("s")
    wid = c * NS + s

    _zero_buf(ones_v, CHUNK, 16)
    for z in range(RPT // CHUNK):
        pltpu.sync_copy(ones_v, deg_sp.at[pl.ds(s * RPT + z * CHUNK, CHUNK)])

    ones16 = jnp.ones((16,), jnp.float32)

    def ones_body(i, _):
        ones_v[i, :] = ones16
        return 0

    lax.fori_loop(0, CHUNK, ones_body, 0)
    plsc.subcore_barrier()

    def blk(q, _):
        q8 = pl.multiple_of(q * BLK, 8)
        pltpu.sync_copy(dst_hbm.at[wid].at[pl.ds(q8, BLK)], dst_v)
        for r in range(BLK):
            pltpu.sync_copy(ones_v, deg_sp.at[dst_v.at[r]], add=True)
        return 0

    lax.fori_loop(0, NBLK, blk, 0)
    plsc.subcore_barrier()

    pltpu.sync_copy(deg_sp.at[pl.ds(s * RPT, RPT)],
                    deg_hbm.at[c].at[pl.ds(s * RPT, RPT)])


_deg_call = pl.kernel(
    _deg_body,
    out_type=jax.ShapeDtypeStruct((NC, NPAD, 16), jnp.float32),
    mesh=_mesh,
    scratch_types=[
        pltpu.VMEM_SHARED((NPAD, 16), jnp.float32),
        pltpu.VMEM((BLK, CHUNK), jnp.int32),
        pltpu.VMEM((CHUNK, 16), jnp.float32),
    ],
)


SUB = 32
NSUB = EPT // SUB
SROWS = EPT // 128


def _agg_body(g_hbm, pk_hbm, out_hbm, acc_sp,
              slab, usrc0, usrc1, udst0, udst1, rows0, rows1, gs0, gs1,
              ss0, ss1):
    c = lax.axis_index("c")
    s = lax.axis_index("s")
    wid = c * NS + s

    usrc = (usrc0, usrc1)
    udst = (udst0, udst1)
    rows = (rows0, rows1)
    gsem = (gs0, gs1)
    ssem = (ss0, ss1)

    _zero_buf(rows0, SUB, D)
    for z in range(RPT // SUB):
        pltpu.sync_copy(rows0, acc_sp.at[pl.ds(s * RPT + z * SUB, SUB)])

    pltpu.sync_copy(pk_hbm.at[wid], slab)
    plsc.subcore_barrier()

    def body(q, _):
        kr0 = 2 * q

        def unpack(jj):
            b = jj % 2
            kr = kr0 + jj // 4
            h = jj % 4
            for i in range(SUB // 16):
                v = slab[kr, pl.ds(32 * h + 16 * i, 16)]
                usrc[b][pl.ds(16 * i, 16)] = lax.shift_right_logical(v, 14)
                udst[b][pl.ds(16 * i, 16)] = lax.bitwise_and(v, 16383)

        def start(jj):
            b = jj % 2
            return pltpu.async_copy(g_hbm.at[usrc[b]], rows[b], gsem[b])

        def scat(jj):
            b = jj % 2
            return pltpu.async_copy(rows[b], acc_sp.at[udst[b]], ssem[b],
                                    add=True)

        for jj in range(8):
            unpack(jj)
            start(jj).wait()
            scat(jj).wait()
        return 0

    lax.fori_loop(0, NSUB // 8, body, 0)
    plsc.subcore_barrier()

    pltpu.sync_copy(acc_sp.at[pl.ds(s * RPT, RPT)],
                    out_hbm.at[c].at[pl.ds(s * RPT, RPT)])


_agg_call = pl.kernel(
    _agg_body,
    out_type=jax.ShapeDtypeStruct((NC, NPAD, D), jnp.float32),
    mesh=_mesh,
    scratch_types=[
        pltpu.VMEM_SHARED((NPAD, D), jnp.float32),
        pltpu.VMEM((SROWS, 128), jnp.int32),
        pltpu.VMEM((SUB,), jnp.int32),
        pltpu.VMEM((SUB,), jnp.int32),
        pltpu.VMEM((SUB,), jnp.int32),
        pltpu.VMEM((SUB,), jnp.int32),
        pltpu.VMEM((SUB, D), jnp.float32),
        pltpu.VMEM((SUB, D), jnp.float32),
        pltpu.SemaphoreType.DMA,
        pltpu.SemaphoreType.DMA,
        pltpu.SemaphoreType.DMA,
        pltpu.SemaphoreType.DMA,
    ],
)


BR = 2000


def _dinv(dga_ref, dgb_ref):
    return lax.rsqrt(dga_ref[:, :1] + dgb_ref[:, :1] + 1.0)


def _k1_body(x_ref, w_ref, dga_ref, dgb_ref, g_ref):
    h = jnp.dot(x_ref[...], w_ref[...], preferred_element_type=jnp.float32)
    g_ref[...] = h * _dinv(dga_ref, dgb_ref)


def _k2_body(sa_ref, sb_ref, g1_ref, dga_ref, dgb_ref, b1_ref, w2_ref, g2_ref):
    dinv = _dinv(dga_ref, dgb_ref)
    y = (sa_ref[...] + sb_ref[...] + g1_ref[...]) * dinv + b1_ref[...]
    y = jnp.maximum(y, 0.0)
    g2_ref[...] = jnp.dot(y, w2_ref[...],
                          preferred_element_type=jnp.float32) * dinv


def _k3_body(sa_ref, sb_ref, g2_ref, dga_ref, dgb_ref, b2_ref, o_ref):
    o_ref[...] = ((sa_ref[...] + sb_ref[...] + g2_ref[...])
                  * _dinv(dga_ref, dgb_ref) + b2_ref[...])


def _row_spec(w):
    return pl.BlockSpec((BR, w), lambda i: (i, 0))


_full_mat = pl.BlockSpec((D, D), lambda i: (0, 0))
_full_vec = pl.BlockSpec((1, D), lambda i: (0, 0))

_k1_call = pl.pallas_call(
    _k1_body,
    grid=(N // BR,),
    in_specs=[_row_spec(D), _full_mat, _row_spec(16), _row_spec(16)],
    out_specs=_row_spec(D),
    out_shape=jax.ShapeDtypeStruct((N, D), jnp.float32),
)

_k2_call = pl.pallas_call(
    _k2_body,
    grid=(N // BR,),
    in_specs=[_row_spec(D), _row_spec(D), _row_spec(D), _row_spec(16),
              _row_spec(16), _full_vec, _full_mat],
    out_specs=_row_spec(D),
    out_shape=jax.ShapeDtypeStruct((N, D), jnp.float32),
)

_k3_call = pl.pallas_call(
    _k3_body,
    grid=(N // BR,),
    in_specs=[_row_spec(D), _row_spec(D), _row_spec(D), _row_spec(16),
              _row_spec(16), _full_vec],
    out_specs=_row_spec(D),
    out_shape=jax.ShapeDtypeStruct((N, D), jnp.float32),
)


def kernel(x, edge_index, W1, b1, W2, b2):
    pad = NW * EPT - E
    src = jnp.concatenate([edge_index[0], jnp.zeros((pad,), jnp.int32)])
    dst = jnp.concatenate([edge_index[1], jnp.full((pad,), N, jnp.int32)])

    packed = ((src << 14) | dst).reshape(NW, SROWS, 128)
    src = src.reshape(NW, NCHUNK, CHUNK)
    dst = dst.reshape(NW, NCHUNK, CHUNK)

    deg = _deg_call(dst)
    dga, dgb = deg[0, :N], deg[1, :N]

    g1 = _k1_call(x, W1, dga, dgb)
    s1 = _agg_call(g1, packed)
    g2 = _k2_call(s1[0, :N], s1[1, :N], g1, dga, dgb, b1.reshape(1, D), W2)
    s2 = _agg_call(g2, packed)
    return _k3_call(s2[0, :N], s2[1, :N], g2, dga, dgb, b2.reshape(1, D))


# speedup vs baseline: 1.2567x; 1.2567x over previous
import jax
import jax.numpy as jnp
from jax import lax
from jax.experimental import pallas as pl
from jax.experimental.pallas import tpu as pltpu
from jax.experimental.pallas import tpu_sc as plsc

N = 10000
E = 320000
D = 128

NC = 2
NS = 16
NW = NC * NS
CHUNK = 80
NCHUNK = 128
EPT = NCHUNK * CHUNK
BLK = 8
NBLK = NCHUNK // BLK
NPAD = 10240
RPT = NPAD // NS

_mesh = plsc.VectorSubcoreMesh(core_axis_name="c", subcore_axis_name="s",
                               num_cores=NC, num_subcores=NS)


def _zero_buf(buf, nrows, ncols):
    zeros16 = jnp.zeros((16,), jnp.float32)

    def body(i, _):
        for j in range(ncols // 16):
            buf[i, pl.ds(j * 16, 16)] = zeros16
        return 0

    lax.fori_loop(0, nrows, body, 0)


def _deg_body(dst_hbm, deg_hbm, deg_sp, dst_v, ones_v):
    c = lax.axis_index("c")
    s = lax.axis_index("s")
    wid = c * NS + s

    _zero_buf(ones_v, CHUNK, 16)
    for z in range(RPT // CHUNK):
        pltpu.sync_copy(ones_v, deg_sp.at[pl.ds(s * RPT + z * CHUNK, CHUNK)])

    ones16 = jnp.ones((16,), jnp.float32)

    def ones_body(i, _):
        ones_v[i, :] = ones16
        return 0

    lax.fori_loop(0, CHUNK, ones_body, 0)
    plsc.subcore_barrier()

    def blk(q, _):
        q8 = pl.multiple_of(q * BLK, 8)
        pltpu.sync_copy(dst_hbm.at[wid].at[pl.ds(q8, BLK)], dst_v)
        for r in range(BLK):
            pltpu.sync_copy(ones_v, deg_sp.at[dst_v.at[r]], add=True)
        return 0

    lax.fori_loop(0, NBLK, blk, 0)
    plsc.subcore_barrier()

    pltpu.sync_copy(deg_sp.at[pl.ds(s * RPT, RPT)],
                    deg_hbm.at[c].at[pl.ds(s * RPT, RPT)])


_deg_call = pl.kernel(
    _deg_body,
    out_type=jax.ShapeDtypeStruct((NC, NPAD, 16), jnp.float32),
    mesh=_mesh,
    scratch_types=[
        pltpu.VMEM_SHARED((NPAD, 16), jnp.float32),
        pltpu.VMEM((BLK, CHUNK), jnp.int32),
        pltpu.VMEM((CHUNK, 16), jnp.float32),
    ],
)


SROWS = EPT // 128


def _agg_body(g_hbm, pk_hbm, out_hbm, acc_sp,
              slab, usrc0, usrc1, udst0, udst1, rows0, rows1, gs0, gs1):
    c = lax.axis_index("c")
    s = lax.axis_index("s")
    wid = c * NS + s

    usrc = (usrc0, usrc1)
    udst = (udst0, udst1)
    rows = (rows0, rows1)
    gsem = (gs0, gs1)

    _zero_buf(rows0, 128, D)
    for z in range(RPT // 128):
        pltpu.sync_copy(rows0, acc_sp.at[pl.ds(s * RPT + z * 128, 128)])

    pltpu.sync_copy(pk_hbm.at[wid], slab)
    plsc.subcore_barrier()

    def unpack(kr, b):
        for i in range(8):
            v = slab[kr, pl.ds(16 * i, 16)]
            usrc[b][pl.ds(16 * i, 16)] = lax.shift_right_logical(v, 14)
            udst[b][pl.ds(16 * i, 16)] = lax.bitwise_and(v, 16383)

    def pair(t, _):
        kr = 2 * t
        unpack(kr, 0)
        d0 = pltpu.async_copy(g_hbm.at[usrc[0]], rows[0], gsem[0])
        unpack(kr + 1, 1)
        d1 = pltpu.async_copy(g_hbm.at[usrc[1]], rows[1], gsem[1])
        d0.wait()
        d1.wait()
        pltpu.sync_copy(rows[0], acc_sp.at[udst[0]], add=True)
        pltpu.sync_copy(rows[1], acc_sp.at[udst[1]], add=True)
        return 0

    lax.fori_loop(0, SROWS // 2, pair, 0)
    plsc.subcore_barrier()

    pltpu.sync_copy(acc_sp.at[pl.ds(s * RPT, RPT)],
                    out_hbm.at[c].at[pl.ds(s * RPT, RPT)])


_agg_call = pl.kernel(
    _agg_body,
    out_type=jax.ShapeDtypeStruct((NC, NPAD, D), jnp.float32),
    mesh=_mesh,
    scratch_types=[
        pltpu.VMEM_SHARED((NPAD, D), jnp.float32),
        pltpu.VMEM((SROWS, 128), jnp.int32),
        pltpu.VMEM((128,), jnp.int32),
        pltpu.VMEM((128,), jnp.int32),
        pltpu.VMEM((128,), jnp.int32),
        pltpu.VMEM((128,), jnp.int32),
        pltpu.VMEM((128, D), jnp.float32),
        pltpu.VMEM((128, D), jnp.float32),
        pltpu.SemaphoreType.DMA,
        pltpu.SemaphoreType.DMA,
    ],
)


BR = 2000


def _dinv(dga_ref, dgb_ref):
    return lax.rsqrt(dga_ref[:, :1] + dgb_ref[:, :1] + 1.0)


def _k1_body(x_ref, w_ref, dga_ref, dgb_ref, g_ref):
    h = jnp.dot(x_ref[...], w_ref[...], preferred_element_type=jnp.float32)
    g_ref[...] = h * _dinv(dga_ref, dgb_ref)


def _k2_body(sa_ref, sb_ref, g1_ref, dga_ref, dgb_ref, b1_ref, w2_ref, g2_ref):
    dinv = _dinv(dga_ref, dgb_ref)
    y = (sa_ref[...] + sb_ref[...] + g1_ref[...]) * dinv + b1_ref[...]
    y = jnp.maximum(y, 0.0)
    g2_ref[...] = jnp.dot(y, w2_ref[...],
                          preferred_element_type=jnp.float32) * dinv


def _k3_body(sa_ref, sb_ref, g2_ref, dga_ref, dgb_ref, b2_ref, o_ref):
    o_ref[...] = ((sa_ref[...] + sb_ref[...] + g2_ref[...])
                  * _dinv(dga_ref, dgb_ref) + b2_ref[...])


def _row_spec(w):
    return pl.BlockSpec((BR, w), lambda i: (i, 0))


_full_mat = pl.BlockSpec((D, D), lambda i: (0, 0))
_full_vec = pl.BlockSpec((1, D), lambda i: (0, 0))

_k1_call = pl.pallas_call(
    _k1_body,
    grid=(N // BR,),
    in_specs=[_row_spec(D), _full_mat, _row_spec(16), _row_spec(16)],
    out_specs=_row_spec(D),
    out_shape=jax.ShapeDtypeStruct((N, D), jnp.float32),
)

_k2_call = pl.pallas_call(
    _k2_body,
    grid=(N // BR,),
    in_specs=[_row_spec(D), _row_spec(D), _row_spec(D), _row_spec(16),
              _row_spec(16), _full_vec, _full_mat],
    out_specs=_row_spec(D),
    out_shape=jax.ShapeDtypeStruct((N, D), jnp.float32),
)

_k3_call = pl.pallas_call(
    _k3_body,
    grid=(N // BR,),
    in_specs=[_row_spec(D), _row_spec(D), _row_spec(D), _row_spec(16),
              _row_spec(16), _full_vec],
    out_specs=_row_spec(D),
    out_shape=jax.ShapeDtypeStruct((N, D), jnp.float32),
)


def kernel(x, edge_index, W1, b1, W2, b2):
    pad = NW * EPT - E
    src = jnp.concatenate([edge_index[0], jnp.zeros((pad,), jnp.int32)])
    dst = jnp.concatenate([edge_index[1], jnp.full((pad,), N, jnp.int32)])

    packed = ((src << 14) | dst).reshape(NW, SROWS, 128)
    src = src.reshape(NW, NCHUNK, CHUNK)
    dst = dst.reshape(NW, NCHUNK, CHUNK)

    deg = _deg_call(dst)
    dga, dgb = deg[0, :N], deg[1, :N]

    g1 = _k1_call(x, W1, dga, dgb)
    s1 = _agg_call(g1, packed)
    g2 = _k2_call(s1[0, :N], s1[1, :N], g1, dga, dgb, b1.reshape(1, D), W2)
    s2 = _agg_call(g2, packed)
    return _k3_call(s2[0, :N], s2[1, :N], g2, dga, dgb, b2.reshape(1, D))


# speedup vs baseline: 3.5646x; 2.8364x over previous
import jax
import jax.numpy as jnp
from jax import lax
from jax.experimental import pallas as pl
from jax.experimental.pallas import tpu as pltpu
from jax.experimental.pallas import tpu_sc as plsc

N = 10000
E = 320000
D = 128

NC = 2
NS = 16
NW = NC * NS
CHUNK = 80
NCHUNK = 128
EPT = NCHUNK * CHUNK
BLK = 8
NBLK = NCHUNK // BLK
NPAD = 10240
RPT = NPAD // NS

_mesh = plsc.VectorSubcoreMesh(core_axis_name="c", subcore_axis_name="s",
                               num_cores=NC, num_subcores=NS)


def _zero_buf(buf, nrows, ncols):
    zeros16 = jnp.zeros((16,), jnp.float32)

    def body(i, _):
        for j in range(ncols // 16):
            buf[i, pl.ds(j * 16, 16)] = zeros16
        return 0

    lax.fori_loop(0, nrows, body, 0)


def _deg_body(dst_hbm, deg_hbm, deg_sp, dst_v, ones_v):
    c = lax.axis_index("c")
    s = lax.axis_index("s")
    wid = c * NS + s

    _zero_buf(ones_v, CHUNK, 16)
    for z in range(RPT // CHUNK):
        pltpu.sync_copy(ones_v, deg_sp.at[pl.ds(s * RPT + z * CHUNK, CHUNK)])

    ones16 = jnp.ones((16,), jnp.float32)

    def ones_body(i, _):
        ones_v[i, :] = ones16
        return 0

    lax.fori_loop(0, CHUNK, ones_body, 0)
    plsc.subcore_barrier()

    def blk(q, _):
        q8 = pl.multiple_of(q * BLK, 8)
        pltpu.sync_copy(dst_hbm.at[wid].at[pl.ds(q8, BLK)], dst_v)
        for r in range(BLK):
            pltpu.sync_copy(ones_v, deg_sp.at[dst_v.at[r]], add=True)
        return 0

    lax.fori_loop(0, NBLK, blk, 0)
    plsc.subcore_barrier()

    pltpu.sync_copy(deg_sp.at[pl.ds(s * RPT, RPT)],
                    deg_hbm.at[c].at[pl.ds(s * RPT, RPT)])


_deg_call = pl.kernel(
    _deg_body,
    out_type=jax.ShapeDtypeStruct((NC, NPAD, 16), jnp.float32),
    mesh=_mesh,
    scratch_types=[
        pltpu.VMEM_SHARED((NPAD, 16), jnp.float32),
        pltpu.VMEM((BLK, CHUNK), jnp.int32),
        pltpu.VMEM((CHUNK, 16), jnp.float32),
    ],
)


SROWS = EPT // 128


def _agg_body(g_hbm, pk_hbm, out_hbm, acc_sp,
              slab, usrc0, usrc1, udst0, udst1, rows0, rows1, gs0, gs1):
    c = lax.axis_index("c")
    s = lax.axis_index("s")
    wid = c * NS + s

    usrc = (usrc0, usrc1)
    udst = (udst0, udst1)
    rows = (rows0, rows1)
    gsem = (gs0, gs1)

    _zero_buf(rows0, 128, D)
    for z in range(RPT // 128):
        pltpu.sync_copy(rows0, acc_sp.at[pl.ds(s * RPT + z * 128, 128)])

    pltpu.sync_copy(pk_hbm.at[wid], slab)
    plsc.subcore_barrier()

    def unpack(kr, b):
        for i in range(8):
            v = slab[kr, pl.ds(16 * i, 16)]
            usrc[b][pl.ds(16 * i, 16)] = lax.shift_right_logical(v, 14)
            udst[b][pl.ds(16 * i, 16)] = lax.bitwise_and(v, 16383)

    def pair(t, _):
        kr = 2 * t
        unpack(kr, 0)
        d0 = pltpu.async_copy(g_hbm.at[usrc[0]], rows[0], gsem[0])
        unpack(kr + 1, 1)
        d1 = pltpu.async_copy(g_hbm.at[usrc[1]], rows[1], gsem[1])
        d0.wait()
        d1.wait()
        pltpu.sync_copy(rows[0], acc_sp.at[udst[0]], add=True)
        pltpu.sync_copy(rows[1], acc_sp.at[udst[1]], add=True)
        return 0

    lax.fori_loop(0, SROWS // 2, pair, 0)
    plsc.subcore_barrier()

    pltpu.sync_copy(acc_sp.at[pl.ds(s * RPT, RPT)],
                    out_hbm.at[c].at[pl.ds(s * RPT, RPT)])


_agg_call = pl.kernel(
    _agg_body,
    out_type=jax.ShapeDtypeStruct((NC, NPAD, D), jnp.float32),
    mesh=_mesh,
    scratch_types=[
        pltpu.VMEM_SHARED((NPAD, D), jnp.float32),
        pltpu.VMEM((SROWS, 128), jnp.int32),
        pltpu.VMEM((128,), jnp.int32),
        pltpu.VMEM((128,), jnp.int32),
        pltpu.VMEM((128,), jnp.int32),
        pltpu.VMEM((128,), jnp.int32),
        pltpu.VMEM((128, D), jnp.float32),
        pltpu.VMEM((128, D), jnp.float32),
        pltpu.SemaphoreType.DMA,
        pltpu.SemaphoreType.DMA,
    ],
)


BR = 2000


def _dinv(dga_ref, dgb_ref):
    return lax.rsqrt(dga_ref[:, :1] + dgb_ref[:, :1] + 1.0)


def _k1_body(x_ref, w_ref, dga_ref, dgb_ref, g_ref):
    h = jnp.dot(x_ref[...], w_ref[...], preferred_element_type=jnp.float32)
    g_ref[...] = h * _dinv(dga_ref, dgb_ref)


def _k2_body(sa_ref, sb_ref, g1_ref, dga_ref, dgb_ref, b1_ref, w2_ref, g2_ref):
    dinv = _dinv(dga_ref, dgb_ref)
    y = (sa_ref[...] + sb_ref[...] + g1_ref[...]) * dinv + b1_ref[...]
    y = jnp.maximum(y, 0.0)
    g2_ref[...] = jnp.dot(y, w2_ref[...],
                          preferred_element_type=jnp.float32) * dinv


def _k3_body(sa_ref, sb_ref, g2_ref, dga_ref, dgb_ref, b2_ref, o_ref):
    o_ref[...] = ((sa_ref[...] + sb_ref[...] + g2_ref[...])
                  * _dinv(dga_ref, dgb_ref) + b2_ref[...])


def _row_spec(w):
    return pl.BlockSpec((BR, w), lambda i: (i, 0))


_full_mat = pl.BlockSpec((D, D), lambda i: (0, 0))
_full_vec = pl.BlockSpec((1, D), lambda i: (0, 0))

_k1_call = pl.pallas_call(
    _k1_body,
    grid=(N // BR,),
    in_specs=[_row_spec(D), _full_mat, _row_spec(16), _row_spec(16)],
    out_specs=_row_spec(D),
    out_shape=jax.ShapeDtypeStruct((N, D), jnp.float32),
)

_k2_call = pl.pallas_call(
    _k2_body,
    grid=(N // BR,),
    in_specs=[_row_spec(D), _row_spec(D), _row_spec(D), _row_spec(16),
              _row_spec(16), _full_vec, _full_mat],
    out_specs=_row_spec(D),
    out_shape=jax.ShapeDtypeStruct((N, D), jnp.float32),
)

_k3_call = pl.pallas_call(
    _k3_body,
    grid=(N // BR,),
    in_specs=[_row_spec(D), _row_spec(D), _row_spec(D), _row_spec(16),
              _row_spec(16), _full_vec],
    out_specs=_row_spec(D),
    out_shape=jax.ShapeDtypeStruct((N, D), jnp.float32),
)


def kernel(x, edge_index, W1, b1, W2, b2):
    pad = NW * EPT - E
    ar = jnp.arange(pad, dtype=jnp.int32)
    src = jnp.concatenate([edge_index[0], ar % N])
    dst = jnp.concatenate([edge_index[1], N + ar % (NPAD - N)])

    packed = ((src << 14) | dst).reshape(NW, SROWS, 128)
    src = src.reshape(NW, NCHUNK, CHUNK)
    dst = dst.reshape(NW, NCHUNK, CHUNK)

    deg = _deg_call(dst)
    dga, dgb = deg[0, :N], deg[1, :N]

    g1 = _k1_call(x, W1, dga, dgb)
    s1 = _agg_call(g1, packed)
    g2 = _k2_call(s1[0, :N], s1[1, :N], g1, dga, dgb, b1.reshape(1, D), W2)
    s2 = _agg_call(g2, packed)
    return _k3_call(s2[0, :N], s2[1, :N], g2, dga, dgb, b2.reshape(1, D))
